# baseline (speedup 1.0000x reference)
import functools

import jax
import jax.numpy as jnp
from jax import lax
from jax.experimental import pallas as pl
from jax.experimental.pallas import tpu as pltpu
from jax.experimental.pallas import tpu_sc as plsc

N = 50000
E = 800000
G = 2000
H = 64

N_PAD = 51200
E_PAD = 819200
G_PAD = 2048

_BN = 3200
_BE = 4096

_f32 = jnp.float32


def _lrelu(x):
    return jnp.where(x >= 0, x, 0.01 * x)


def _dot_t(x, w):
    return lax.dot_general(x, w, (((1,), (1,)), ((), ())),
                           preferred_element_type=_f32)


def _rowdot(x, a):
    return jnp.sum(x * a, axis=1, keepdims=True)


def _row_spec(bm, n):
    return pl.BlockSpec((bm, n), lambda i: (i, 0))


def _full_spec(a, b):
    return pl.BlockSpec((a, b), lambda i: (0, 0))


def _emb_body(x_ref, w_ref, b_ref, o_ref):
    o_ref[...] = _lrelu(_dot_t(x_ref[...], w_ref[...]) + b_ref[...])


def _tc_embed(x, w, b, bm):
    m, k = x.shape
    h = w.shape[0]
    return pl.pallas_call(
        _emb_body,
        grid=(m // bm,),
        in_specs=[_row_spec(bm, k), _full_spec(h, k), _full_spec(1, h)],
        out_specs=_row_spec(bm, h),
        out_shape=jax.ShapeDtypeStruct((m, h), _f32),
    )(x, w, b)


def _node_body(x_ref, wn_ref, bn_ref, w1_ref, a2_ref, h_ref, hw1_ref, hda_ref):
    h = _lrelu(_dot_t(x_ref[...], wn_ref[...]) + bn_ref[...])
    h_ref[...] = h
    hw1_ref[...] = _dot_t(h, w1_ref[...])
    hda_ref[...] = _rowdot(h, a2_ref[...])


def _tc_node(x, wn, bn, w1, a2):
    return pl.pallas_call(
        _node_body,
        grid=(N_PAD // _BN,),
        in_specs=[_row_spec(_BN, H), _full_spec(H, H), _full_spec(1, H),
                  _full_spec(H, H), _full_spec(1, H)],
        out_specs=[_row_spec(_BN, H), _row_spec(_BN, H), _row_spec(_BN, 1)],
        out_shape=[jax.ShapeDtypeStruct((N_PAD, H), _f32),
                   jax.ShapeDtypeStruct((N_PAD, H), _f32),
                   jax.ShapeDtypeStruct((N_PAD, 1), _f32)],
    )(x, wn, bn, w1, a2)


def _edge_body(hs_ref, er_ref, ew_ref, eb0_ref, w2_ref, eb_ref, a1_ref,
               ab_ref, nm_ref, s1_ref):
    e = _lrelu(_dot_t(er_ref[...], ew_ref[...]) + eb0_ref[...])
    nm = _lrelu(hs_ref[...] + _dot_t(e, w2_ref[...]) + eb_ref[...])
    nm_ref[...] = nm
    s1_ref[...] = _rowdot(nm, a1_ref[...]) + ab_ref[0, 0]


def _tc_edge(hsrc, eraw, ew, eb0, w2, eb, a1, ab):
    return pl.pallas_call(
        _edge_body,
        grid=(E_PAD // _BE,),
        in_specs=[_row_spec(_BE, H), _row_spec(_BE, 16), _full_spec(H, 16),
                  _full_spec(1, H), _full_spec(H, H),
                  _full_spec(1, H), _full_spec(1, H), _full_spec(1, 1)],
        out_specs=[_row_spec(_BE, H), _row_spec(_BE, 1)],
        out_shape=[jax.ShapeDtypeStruct((E_PAD, H), _f32),
                   jax.ShapeDtypeStruct((E_PAD, 1), _f32)],
    )(hsrc, eraw, ew, eb0, w2, eb, a1, ab)


def _attc_body(nm_ref, att_ref, w_ref, b_ref, oa_ref, ob_ref):
    attn = _dot_t(nm_ref[...], w_ref[...]) + b_ref[...]
    attc = att_ref[...] * attn
    oa_ref[...] = attc[:, :H // 2]
    ob_ref[...] = attc[:, H // 2:]


def _tc_attc(nm, att, w, b, bm):
    m = nm.shape[0]
    return pl.pallas_call(
        _attc_body,
        grid=(m // bm,),
        in_specs=[_row_spec(bm, H), _row_spec(bm, 1), _full_spec(H, H),
                  _full_spec(1, H)],
        out_specs=[_row_spec(bm, H // 2), _row_spec(bm, H // 2)],
        out_shape=[jax.ShapeDtypeStruct((m, H // 2), _f32),
                   jax.ShapeDtypeStruct((m, H // 2), _f32)],
    )(nm, att, w, b)


def _gru_body(ca_ref, cb_ref, h_ref, wih_ref, whh_ref, bih_ref, bhh_ref,
              o_ref):
    ctx = jnp.concatenate([ca_ref[...], cb_ref[...]], axis=1)
    ctx = jnp.where(ctx > 0, ctx, jnp.exp(ctx) - 1.0)
    h = h_ref[...]
    gi = _dot_t(ctx, wih_ref[...]) + bih_ref[...]
    gh = _dot_t(h, whh_ref[...]) + bhh_ref[...]
    r = jax.nn.sigmoid(gi[:, :H] + gh[:, :H])
    z = jax.nn.sigmoid(gi[:, H:2 * H] + gh[:, H:2 * H])
    n = jnp.tanh(gi[:, 2 * H:] + r * gh[:, 2 * H:])
    o_ref[...] = jnp.maximum((1.0 - z) * n + z * h, 0.0)


def _tc_gru(ca, cb, h, g, bm):
    m = ca.shape[0]
    return pl.pallas_call(
        _gru_body,
        grid=(m // bm,),
        in_specs=[_row_spec(bm, H // 2), _row_spec(bm, H // 2),
                  _row_spec(bm, H), _full_spec(3 * H, H), _full_spec(3 * H, H),
                  _full_spec(1, 3 * H), _full_spec(1, 3 * H)],
        out_specs=_row_spec(bm, H),
        out_shape=jax.ShapeDtypeStruct((m, H), _f32),
    )(ca, cb, h, g['Wih'], g['Whh'], g['bih'].reshape(1, -1),
      g['bhh'].reshape(1, -1))


_BG = 1024


def _onehot(gid_ref):
    g = jax.lax.broadcasted_iota(jnp.int32, (_BG, G_PAD), 1)
    return (gid_ref[...] == g).astype(_f32)


def _gseg_body(vals_ref, gid_ref, oa_ref, ob_ref):
    i = pl.program_id(0)
    m = lax.dot_general(_onehot(gid_ref), vals_ref[...],
                        (((0,), (0,)), ((), ())),
                        preferred_element_type=_f32)

    @pl.when(i == 0)
    def _():
        oa_ref[...] = m[:, :H // 2]
        ob_ref[...] = m[:, H // 2:]

    @pl.when(i > 0)
    def _():
        oa_ref[...] += m[:, :H // 2]
        ob_ref[...] += m[:, H // 2:]


def _tc_gseg(vals, gid2d):
    return pl.pallas_call(
        _gseg_body,
        grid=(N_PAD // _BG,),
        in_specs=[_row_spec(_BG, H), _row_spec(_BG, 1)],
        out_specs=[_full_spec(G_PAD, H // 2), _full_spec(G_PAD, H // 2)],
        out_shape=[jax.ShapeDtypeStruct((G_PAD, H // 2), _f32),
                   jax.ShapeDtypeStruct((G_PAD, H // 2), _f32)],
    )(vals, gid2d)


def _molsum_body(s1_ref, ga_ref, gid_ref, w_ref, sums_ref):
    i = pl.program_id(0)
    oh = _onehot(gid_ref)
    gag = jnp.sum(oh * ga_ref[...], axis=1, keepdims=True)
    sc = s1_ref[...] + gag
    w = jnp.exp(jnp.where(sc >= 0, sc, 0.01 * sc))
    w_ref[...] = w
    contrib = jnp.sum(oh * w, axis=0).reshape(1, G_PAD)

    @pl.when(i == 0)
    def _():
        sums_ref[...] = contrib

    @pl.when(i > 0)
    def _():
        sums_ref[...] += contrib


def _tc_molsum(s1, ga_row, gid2d):
    return pl.pallas_call(
        _molsum_body,
        grid=(N_PAD // _BG,),
        in_specs=[_row_spec(_BG, 1), _full_spec(1, G_PAD), _row_spec(_BG, 1)],
        out_specs=[_row_spec(_BG, 1), _full_spec(1, G_PAD)],
        out_shape=[jax.ShapeDtypeStruct((N_PAD, 1), _f32),
                   jax.ShapeDtypeStruct((1, G_PAD), _f32)],
    )(s1, ga_row, gid2d)


def _molctx_body(w_ref, sums_ref, gid_ref, hn_ref, oa_ref, ob_ref):
    i = pl.program_id(0)
    oh = _onehot(gid_ref)
    sg = jnp.sum(oh * sums_ref[...], axis=1, keepdims=True)
    aw = w_ref[...] / (sg + 1e-12)
    m = lax.dot_general(oh, aw * hn_ref[...], (((0,), (0,)), ((), ())),
                        preferred_element_type=_f32)

    @pl.when(i == 0)
    def _():
        oa_ref[...] = m[:, :H // 2]
        ob_ref[...] = m[:, H // 2:]

    @pl.when(i > 0)
    def _():
        oa_ref[...] += m[:, :H // 2]
        ob_ref[...] += m[:, H // 2:]


def _tc_molctx(w, sums_row, gid2d, hn):
    return pl.pallas_call(
        _molctx_body,
        grid=(N_PAD // _BG,),
        in_specs=[_row_spec(_BG, 1), _full_spec(1, G_PAD), _row_spec(_BG, 1),
                  _row_spec(_BG, H)],
        out_specs=[_full_spec(G_PAD, H // 2), _full_spec(G_PAD, H // 2)],
        out_shape=[jax.ShapeDtypeStruct((G_PAD, H // 2), _f32),
                   jax.ShapeDtypeStruct((G_PAD, H // 2), _f32)],
    )(w, sums_row, gid2d, hn)


def _molg_body(s_ref, a2_ref, sa_ref, ga_ref):
    sa = _lrelu(s_ref[...])
    sa_ref[...] = sa
    ga_ref[...] = _rowdot(sa, a2_ref[...])


def _tc_molg(s, a2):
    return pl.pallas_call(
        _molg_body,
        grid=(1,),
        in_specs=[_row_spec(G_PAD, H), _full_spec(1, H)],
        out_specs=[_row_spec(G_PAD, H), _row_spec(G_PAD, 1)],
        out_shape=[jax.ShapeDtypeStruct((G_PAD, H), _f32),
                   jax.ShapeDtypeStruct((G_PAD, 1), _f32)],
    )(s, a2)


def _moln_body(x_ref, a1_ref, ab_ref, w_ref, b_ref, s1_ref, hn_ref):
    x = x_ref[...]
    s1_ref[...] = _rowdot(x, a1_ref[...]) + ab_ref[0, 0]
    hn_ref[...] = _dot_t(x, w_ref[...]) + b_ref[...]


def _tc_moln(x, a1, ab, w, b):
    return pl.pallas_call(
        _moln_body,
        grid=(N_PAD // _BN,),
        in_specs=[_row_spec(_BN, H), _full_spec(1, H), _full_spec(1, 1),
                  _full_spec(H, H), _full_spec(1, H)],
        out_specs=[_row_spec(_BN, 1), _row_spec(_BN, H)],
        out_shape=[jax.ShapeDtypeStruct((N_PAD, 1), _f32),
                   jax.ShapeDtypeStruct((N_PAD, H), _f32)],
    )(x, a1, ab, w, b)


def _pred_body(s_ref, w_ref, b_ref, o_ref):
    o_ref[...] = _rowdot(s_ref[...], w_ref[...]) + b_ref[0, 0]


def _tc_pred(s, w, b):
    return pl.pallas_call(
        _pred_body,
        grid=(1,),
        in_specs=[_row_spec(G_PAD, H), _full_spec(1, H), _full_spec(1, 1)],
        out_specs=_row_spec(G_PAD, 1),
        out_shape=jax.ShapeDtypeStruct((G_PAD, 1), _f32),
    )(s, w, b)


_MESH = plsc.VectorSubcoreMesh(core_axis_name="c", subcore_axis_name="s")
_SC_PARAMS = pltpu.CompilerParams(use_tc_tiling_on_sc=False,
                                  needs_layout_passes=False)
_NC = 2
_NS = 16
_NW = _NC * _NS
_C = 128
_GR = 512


def _gather_rows(table, idx2):
    rows = _GR // _C
    per_w = E_PAD // _NW
    n_chunks = per_w // _GR
    n_pairs = n_chunks // 2

    @functools.partial(
        pl.kernel,
        mesh=_MESH,
        out_type=jax.ShapeDtypeStruct((E_PAD // _C, _C, H), _f32),
        scratch_types=[
            pltpu.VMEM((rows, _C), jnp.int32),
            pltpu.VMEM((rows, _C), jnp.int32),
            pltpu.VMEM((rows, _C, H), _f32),
            pltpu.VMEM((rows, _C, H), _f32),
            pltpu.SemaphoreType.DMA,
            pltpu.SemaphoreType.DMA,
        ],
        compiler_params=_SC_PARAMS,
    )
    def k(table_hbm, idx_hbm, out_hbm, i0, i1, r0, r1, s0, s1):
        wid = lax.axis_index("s") * _NC + lax.axis_index("c")
        ibase = wid * (per_w // _C)
        idx_b = (i0, i1)
        row_b = (r0, r1)
        sem_b = (s0, s1)

        def fire(b, crow):
            pltpu.sync_copy(idx_hbm.at[pl.ds(crow, rows)], idx_b[b])
            for j in range(rows):
                pltpu.async_copy(table_hbm.at[idx_b[b].at[j]],
                                 row_b[b].at[j], sem_b[b])

        def drain(b):
            for j in range(rows):
                pltpu.make_async_copy(table_hbm.at[idx_b[b].at[j]],
                                      row_b[b].at[j], sem_b[b]).wait()

        for b in range(2):
            fire(b, ibase + b * rows)

        def body(i, carry):
            for b in range(2):
                c = 2 * i + b
                crow = ibase + c * rows
                drain(b)
                pltpu.sync_copy(row_b[b], out_hbm.at[pl.ds(crow, rows)])

                @pl.when(c + 2 < n_chunks)
                def _():
                    fire(b, crow + 2 * rows)
            return carry

        lax.fori_loop(0, n_pairs, body, 0)

    return k(table, idx2)


def _seg_softmax_sc(s1_2d, table, idx2, zeros, m_sz, k_sz, rows):
    m_rows = m_sz // _C
    per_sub = m_rows // _NS
    n_chunks = per_sub // rows

    @functools.partial(
        pl.kernel,
        mesh=_MESH,
        out_type=jax.ShapeDtypeStruct((m_rows, _C), _f32),
        scratch_types=[
            pltpu.VMEM_SHARED((k_sz,), _f32),
            pltpu.VMEM((k_sz,), _f32),
            pltpu.VMEM((k_sz,), _f32),
            pltpu.VMEM((rows, _C), jnp.int32),
            pltpu.VMEM((rows, _C), _f32),
            pltpu.VMEM((rows, _C), _f32),
            pltpu.SemaphoreType.DMA,
        ],
        compiler_params=_SC_PARAMS,
    )
    def k(s1_hbm, tab_hbm, idx_hbm, z_hbm, out_hbm,
          spsum, tab_v, sums_v, idx_v, s1_v, w_v, sem):
        c = lax.axis_index("c")
        s = lax.axis_index("s")
        pltpu.sync_copy(tab_hbm, tab_v)

        @pl.when(s == 0)
        def _():
            pltpu.sync_copy(z_hbm, spsum)

        plsc.subcore_barrier()

        def _chunk_w(roff):
            pltpu.sync_copy(idx_hbm.at[pl.ds(roff, rows)], idx_v)
            pltpu.sync_copy(s1_hbm.at[pl.ds(roff, rows)], s1_v)
            for j in range(rows):
                for q in range(_C // 16):
                    sl = pl.ds(q * 16, 16)
                    iv = idx_v[j, sl]
                    tv = plsc.load_gather(tab_v, [iv])
                    sc = s1_v[j, sl] + tv
                    sc = jnp.where(sc >= 0, sc, 0.01 * sc)
                    w_v[j, sl] = jnp.exp(sc)

        def ph1(i, carry):
            roff = s * per_sub + i * rows
            _chunk_w(roff)
            for j in range(rows):
                pltpu.async_copy(w_v.at[j], spsum.at[idx_v.at[j]], sem,
                                 add=True)
            for j in range(rows):
                pltpu.make_async_copy(w_v.at[j], spsum.at[idx_v.at[j]],
                                      sem).wait()
            return carry

        lax.fori_loop(0, n_chunks, ph1, 0)
        plsc.subcore_barrier()
        pltpu.sync_copy(spsum, sums_v)

        def ph2(i, carry):
            roff = s * per_sub + (2 * i + c) * rows
            _chunk_w(roff)
            for j in range(rows):
                for q in range(_C // 16):
                    sl = pl.ds(q * 16, 16)
                    iv = idx_v[j, sl]
                    sg = plsc.load_gather(sums_v, [iv])
                    w_v[j, sl] = w_v[j, sl] / (sg + 1e-12)
            pltpu.sync_copy(w_v, out_hbm.at[pl.ds(roff, rows)])
            return carry

        lax.fori_loop(0, (n_chunks + 1 - c) // 2, ph2, 0)

    return k(s1_2d, table, idx2, zeros)


def _seg_sum32(vals_a, vals_b, idx2, zeros32, m_sz, k_sz, rows):
    chunk = rows * _C
    per_sub = m_sz // _NS
    n_chunks = per_sub // chunk
    rows_out = k_sz // _NS

    @functools.partial(
        pl.kernel,
        mesh=_MESH,
        out_type=[jax.ShapeDtypeStruct((k_sz, H // 2), _f32),
                  jax.ShapeDtypeStruct((k_sz, H // 2), _f32)],
        scratch_types=[
            pltpu.VMEM_SHARED((k_sz, H // 2), _f32),
            pltpu.VMEM((rows, _C), jnp.int32),
            pltpu.VMEM((rows, _C, H // 2), _f32),
            pltpu.SemaphoreType.DMA,
        ],
        compiler_params=_SC_PARAMS,
    )
    def k(va_hbm, vb_hbm, idx_hbm, z_hbm, oa_hbm, ob_hbm,
          spacc, idx_v, vals_v, sem):
        c = lax.axis_index("c")
        s = lax.axis_index("s")

        @pl.when(s == 0)
        def _():
            pltpu.sync_copy(z_hbm, spacc)

        plsc.subcore_barrier()

        def body(i, carry):
            roff = s * (per_sub // _C) + i * rows
            pltpu.sync_copy(idx_hbm.at[pl.ds(roff, rows)], idx_v)

            @pl.when(c == 0)
            def _():
                pltpu.sync_copy(va_hbm.at[pl.ds(roff, rows)], vals_v)

            @pl.when(c == 1)
            def _():
                pltpu.sync_copy(vb_hbm.at[pl.ds(roff, rows)], vals_v)

            for j in range(rows):
                pltpu.async_copy(vals_v.at[j], spacc.at[idx_v.at[j]], sem,
                                 add=True)
            for j in range(rows):
                pltpu.make_async_copy(vals_v.at[j], spacc.at[idx_v.at[j]],
                                      sem).wait()
            return carry

        lax.fori_loop(0, n_chunks, body, 0)
        plsc.subcore_barrier()
        o = s * rows_out

        @pl.when(c == 0)
        def _():
            pltpu.sync_copy(spacc.at[pl.ds(o, rows_out)],
                            oa_hbm.at[pl.ds(o, rows_out)])

        @pl.when(c == 1)
        def _():
            pltpu.sync_copy(spacc.at[pl.ds(o, rows_out)],
                            ob_hbm.at[pl.ds(o, rows_out)])

    return k(vals_a, vals_b, idx2, zeros32)


def kernel(node, edge, edge_index, node_graph_ids, params):
    f32 = _f32
    node_p = jnp.zeros((N_PAD, H), f32).at[:N, :node.shape[1]].set(node)
    edge_p = jnp.zeros((E_PAD, 16), f32).at[:E, :edge.shape[1]].set(edge)
    src = jnp.clip(edge_index[0].astype(jnp.int32), 0, N - 1)
    src2 = jnp.zeros((E_PAD,), jnp.int32).at[:E].set(src).reshape(
        E_PAD // _C, _C)
    dst2 = jnp.full((E_PAD,), N, jnp.int32).at[:E].set(
        edge_index[1].astype(jnp.int32)).reshape(E_PAD // _C, _C)
    gid2d = jnp.full((N_PAD,), G, jnp.int32).at[:N].set(
        node_graph_ids.astype(jnp.int32)).reshape(N_PAD, 1)

    zN = jnp.zeros((N_PAD,), f32)
    zN32 = jnp.zeros((N_PAD, H // 2), f32)

    embN_W = jnp.zeros((H, H), f32).at[:, :node.shape[1]].set(params['embN_W'])
    embE_W = jnp.zeros((H, 16), f32).at[:, :edge.shape[1]].set(params['embE_W'])

    x = _tc_embed(node_p, embN_W, params['embN_b'].reshape(1, H), _BN)
    embE_b = params['embE_b'].reshape(1, H)

    for p in params['atom']:
        w1 = p['edge_W'][:, :H]
        w2 = p['edge_W'][:, H:]
        a1 = p['align_W'][:, :H]
        a2 = p['align_W'][:, H:]
        ab = p['align_b'].reshape(1, 1)
        h, hW1, hda = _tc_node(x, p['node_W'], p['node_b'].reshape(1, H),
                               w1, a2)
        hsrc = _gather_rows(hW1, src2).reshape(E_PAD, H)
        nm, s1 = _tc_edge(hsrc, edge_p, embE_W, embE_b, w2,
                          p['edge_b'].reshape(1, H), a1, ab)
        att = _seg_softmax_sc(s1.reshape(E_PAD // _C, _C),
                              hda.reshape(N_PAD), dst2, zN, E_PAD, N_PAD, 8)
        attcA, attcB = _tc_attc(nm, att.reshape(E_PAD, 1), p['attend_W'],
                                p['attend_b'].reshape(1, H), _BE)
        cA, cB = _seg_sum32(attcA.reshape(E_PAD // _C, _C, H // 2),
                            attcB.reshape(E_PAD // _C, _C, H // 2),
                            dst2, zN32, E_PAD, N_PAD, 4)
        x = _tc_gru(cA, cB, h, p['gru'], _BN)

    sA, sB = _tc_gseg(x, gid2d)
    s = jnp.concatenate([sA, sB], axis=1)

    for p in params['mol']:
        a1 = p['align_W'][:, :H]
        a2 = p['align_W'][:, H:]
        ab = p['align_b'].reshape(1, 1)
        sa, ga = _tc_molg(s, a2)
        s1m, hn = _tc_moln(x, a1, ab, p['attend_W'],
                           p['attend_b'].reshape(1, H))
        w, sums = _tc_molsum(s1m, ga.reshape(1, G_PAD), gid2d)
        cA, cB = _tc_molctx(w, sums, gid2d, hn)
        s = _tc_gru(cA, cB, sa, p['gru'], G_PAD)

    out = _tc_pred(s, params['pred_W'], params['pred_b'].reshape(1, 1))
    return out[:G]


# speedup vs baseline: 1.0205x; 1.0205x over previous
import functools

import jax
import jax.numpy as jnp
from jax import lax
from jax.experimental import pallas as pl
from jax.experimental.pallas import tpu as pltpu
from jax.experimental.pallas import tpu_sc as plsc

N = 50000
E = 800000
G = 2000
H = 64

N_PAD = 51200
E_PAD = 819200
G_PAD = 2048

_BN = 3200
_BE = 4096

_f32 = jnp.float32


def _lrelu(x):
    return jnp.where(x >= 0, x, 0.01 * x)


def _dot_t(x, w):
    return lax.dot_general(x, w, (((1,), (1,)), ((), ())),
                           preferred_element_type=_f32)


def _rowdot(x, a):
    return jnp.sum(x * a, axis=1, keepdims=True)


def _row_spec(bm, n):
    return pl.BlockSpec((bm, n), lambda i: (i, 0))


def _full_spec(a, b):
    return pl.BlockSpec((a, b), lambda i: (0, 0))


def _emb_body(x_ref, w_ref, b_ref, o_ref):
    o_ref[...] = _lrelu(_dot_t(x_ref[...], w_ref[...]) + b_ref[...])


def _tc_embed(x, w, b, bm):
    m, k = x.shape
    h = w.shape[0]
    return pl.pallas_call(
        _emb_body,
        grid=(m // bm,),
        in_specs=[_row_spec(bm, k), _full_spec(h, k), _full_spec(1, h)],
        out_specs=_row_spec(bm, h),
        out_shape=jax.ShapeDtypeStruct((m, h), _f32),
    )(x, w, b)


def _node_out(h, w1_ref, a2_ref, h_ref, hw1_ref, hda_ref):
    h_ref[...] = h
    hw1_ref[...] = _dot_t(h, w1_ref[...])
    hda_ref[...] = _rowdot(h, a2_ref[...])


def _embnode_body(x_ref, we_ref, be_ref, wn_ref, bn_ref, w1_ref, a2_ref,
                  h_ref, hw1_ref, hda_ref):
    x = _lrelu(_dot_t(x_ref[...], we_ref[...]) + be_ref[...])
    h = _lrelu(_dot_t(x, wn_ref[...]) + bn_ref[...])
    _node_out(h, w1_ref, a2_ref, h_ref, hw1_ref, hda_ref)


def _tc_embnode(node_p, we, be, wn, bn, w1, a2):
    return pl.pallas_call(
        _embnode_body,
        grid=(N_PAD // _BN,),
        in_specs=[_row_spec(_BN, H), _full_spec(H, H), _full_spec(1, H),
                  _full_spec(H, H), _full_spec(1, H),
                  _full_spec(H, H), _full_spec(1, H)],
        out_specs=[_row_spec(_BN, H), _row_spec(_BN, H), _row_spec(_BN, 1)],
        out_shape=[jax.ShapeDtypeStruct((N_PAD, H), _f32),
                   jax.ShapeDtypeStruct((N_PAD, H), _f32),
                   jax.ShapeDtypeStruct((N_PAD, 1), _f32)],
    )(node_p, we, be, wn, bn, w1, a2)


def _gru_math(ca, cb, h, wih_ref, whh_ref, bih_ref, bhh_ref):
    ctx = jnp.concatenate([ca, cb], axis=1)
    ctx = jnp.where(ctx > 0, ctx, jnp.exp(ctx) - 1.0)
    gi = _dot_t(ctx, wih_ref[...]) + bih_ref[...]
    gh = _dot_t(h, whh_ref[...]) + bhh_ref[...]
    r = jax.nn.sigmoid(gi[:, :H] + gh[:, :H])
    z = jax.nn.sigmoid(gi[:, H:2 * H] + gh[:, H:2 * H])
    n = jnp.tanh(gi[:, 2 * H:] + r * gh[:, 2 * H:])
    return jnp.maximum((1.0 - z) * n + z * h, 0.0)


def _grunode_body(ca_ref, cb_ref, h_ref, wih_ref, whh_ref, bih_ref, bhh_ref,
                  wn_ref, bn_ref, w1_ref, a2_ref, h2_ref, hw1_ref, hda_ref):
    x = _gru_math(ca_ref[...], cb_ref[...], h_ref[...],
                  wih_ref, whh_ref, bih_ref, bhh_ref)
    h2 = _lrelu(_dot_t(x, wn_ref[...]) + bn_ref[...])
    _node_out(h2, w1_ref, a2_ref, h2_ref, hw1_ref, hda_ref)


def _tc_grunode(ca, cb, h, g, wn, bn, w1, a2):
    return pl.pallas_call(
        _grunode_body,
        grid=(N_PAD // _BN,),
        in_specs=[_row_spec(_BN, H // 2), _row_spec(_BN, H // 2),
                  _row_spec(_BN, H), _full_spec(3 * H, H),
                  _full_spec(3 * H, H), _full_spec(1, 3 * H),
                  _full_spec(1, 3 * H), _full_spec(H, H), _full_spec(1, H),
                  _full_spec(H, H), _full_spec(1, H)],
        out_specs=[_row_spec(_BN, H), _row_spec(_BN, H), _row_spec(_BN, 1)],
        out_shape=[jax.ShapeDtypeStruct((N_PAD, H), _f32),
                   jax.ShapeDtypeStruct((N_PAD, H), _f32),
                   jax.ShapeDtypeStruct((N_PAD, 1), _f32)],
    )(ca, cb, h, g['Wih'], g['Whh'], g['bih'].reshape(1, -1),
      g['bhh'].reshape(1, -1), wn, bn, w1, a2)


def _edge_body(hs_ref, er_ref, ew_ref, eb0_ref, w2_ref, eb_ref, a1_ref,
               ab_ref, nm_ref, s1_ref):
    e = _lrelu(_dot_t(er_ref[...], ew_ref[...]) + eb0_ref[...])
    nm = _lrelu(hs_ref[...] + _dot_t(e, w2_ref[...]) + eb_ref[...])
    nm_ref[...] = nm
    s1_ref[...] = _rowdot(nm, a1_ref[...]) + ab_ref[0, 0]


def _tc_edge(hsrc, eraw, ew, eb0, w2, eb, a1, ab):
    return pl.pallas_call(
        _edge_body,
        grid=(E_PAD // _BE,),
        in_specs=[_row_spec(_BE, H), _row_spec(_BE, 16), _full_spec(H, 16),
                  _full_spec(1, H), _full_spec(H, H),
                  _full_spec(1, H), _full_spec(1, H), _full_spec(1, 1)],
        out_specs=[_row_spec(_BE, H), _row_spec(_BE, 1)],
        out_shape=[jax.ShapeDtypeStruct((E_PAD, H), _f32),
                   jax.ShapeDtypeStruct((E_PAD, 1), _f32)],
    )(hsrc, eraw, ew, eb0, w2, eb, a1, ab)


def _attc_body(nm_ref, att_ref, w_ref, b_ref, oa_ref, ob_ref):
    attn = _dot_t(nm_ref[...], w_ref[...]) + b_ref[...]
    attc = att_ref[...] * attn
    oa_ref[...] = attc[:, :H // 2]
    ob_ref[...] = attc[:, H // 2:]


def _tc_attc(nm, att, w, b, bm):
    m = nm.shape[0]
    return pl.pallas_call(
        _attc_body,
        grid=(m // bm,),
        in_specs=[_row_spec(bm, H), _row_spec(bm, 1), _full_spec(H, H),
                  _full_spec(1, H)],
        out_specs=[_row_spec(bm, H // 2), _row_spec(bm, H // 2)],
        out_shape=[jax.ShapeDtypeStruct((m, H // 2), _f32),
                   jax.ShapeDtypeStruct((m, H // 2), _f32)],
    )(nm, att, w, b)


def _gru_body(ca_ref, cb_ref, h_ref, wih_ref, whh_ref, bih_ref, bhh_ref,
              o_ref):
    o_ref[...] = _gru_math(ca_ref[...], cb_ref[...], h_ref[...],
                           wih_ref, whh_ref, bih_ref, bhh_ref)


def _tc_gru(ca, cb, h, g, bm):
    m = ca.shape[0]
    return pl.pallas_call(
        _gru_body,
        grid=(m // bm,),
        in_specs=[_row_spec(bm, H // 2), _row_spec(bm, H // 2),
                  _row_spec(bm, H), _full_spec(3 * H, H), _full_spec(3 * H, H),
                  _full_spec(1, 3 * H), _full_spec(1, 3 * H)],
        out_specs=_row_spec(bm, H),
        out_shape=jax.ShapeDtypeStruct((m, H), _f32),
    )(ca, cb, h, g['Wih'], g['Whh'], g['bih'].reshape(1, -1),
      g['bhh'].reshape(1, -1))


_BG = 512
_NBG = N_PAD // _BG


def _molphase_body(x_ref, gid_ref, mp_refs, pw_ref, pb_ref, o_ref, w_ref):
    def onehot_t(b):
        gid = gid_ref[pl.ds(b, 1), :]
        g = jax.lax.broadcasted_iota(jnp.int32, (G_PAD, _BG), 0)
        return (gid == g).astype(_f32)

    def xblk(b):
        return x_ref[pl.ds(b * _BG, _BG), :]

    def seg_x(b, acc):
        return acc + lax.dot_general(
            onehot_t(b), xblk(b),
            (((1,), (0,)), ((), ())), preferred_element_type=_f32)

    s = lax.fori_loop(0, _NBG, seg_x, jnp.zeros((G_PAD, H), _f32))

    for (a1, a2, ab, aw_w, aw_b, wih, whh, bih, bhh) in mp_refs:
        sa = _lrelu(s)
        ga_col = _rowdot(sa, a2[...])

        def sums_step(b, acc):
            oh = onehot_t(b)
            s1_row = lax.dot_general(a1[...], xblk(b), (((1,), (1,)), ((), ())),
                                     preferred_element_type=_f32) + ab[0, 0]
            gag = jnp.sum(oh * ga_col, axis=0, keepdims=True)
            sc = s1_row + gag
            w = jnp.exp(jnp.where(sc >= 0, sc, 0.01 * sc))
            w_ref[pl.ds(b, 1), :] = w
            return acc + jnp.sum(oh * w, axis=1, keepdims=True)

        sums = lax.fori_loop(0, _NBG, sums_step,
                             jnp.zeros((G_PAD, 1), _f32))

        def ctx_step(b, acc):
            oh = onehot_t(b)
            hn = _dot_t(xblk(b), aw_w[...]) + aw_b[...]
            sg = jnp.sum(oh * sums, axis=0, keepdims=True)
            aw = w_ref[pl.ds(b, 1), :] / (sg + 1e-12)
            return acc + lax.dot_general(oh * aw, hn,
                                         (((1,), (0,)), ((), ())),
                                         preferred_element_type=_f32)

        ctx = lax.fori_loop(0, _NBG, ctx_step, jnp.zeros((G_PAD, H), _f32))
        s = _gru_math(ctx[:, :H // 2], ctx[:, H // 2:], sa,
                      wih, whh, bih, bhh)

    o_ref[...] = _rowdot(s, pw_ref[...]) + pb_ref[0, 0]


def _tc_molphase(x, gid2d, mol_params, pred_w, pred_b):
    flat = []
    for p in mol_params:
        flat += [p['align_W'][:, :H], p['align_W'][:, H:],
                 p['align_b'].reshape(1, 1), p['attend_W'],
                 p['attend_b'].reshape(1, H), p['gru']['Wih'],
                 p['gru']['Whh'], p['gru']['bih'].reshape(1, -1),
                 p['gru']['bhh'].reshape(1, -1)]

    def body(x_ref, gid_ref, *rest):
        n = 9
        mp_refs = [tuple(rest[i * n:(i + 1) * n]) for i in range(len(mol_params))]
        k = len(mol_params) * n
        _molphase_body(x_ref, gid_ref, mp_refs, rest[k], rest[k + 1],
                       rest[k + 2], rest[k + 3])

    in_specs = [_full_spec(N_PAD, H), _full_spec(N_PAD // _BG, _BG)]
    for p in mol_params:
        in_specs += [_full_spec(1, H), _full_spec(1, H), _full_spec(1, 1),
                     _full_spec(H, H), _full_spec(1, H),
                     _full_spec(3 * H, H), _full_spec(3 * H, H),
                     _full_spec(1, 3 * H), _full_spec(1, 3 * H)]
    in_specs += [_full_spec(1, H), _full_spec(1, 1)]
    return pl.pallas_call(
        body,
        grid=(1,),
        in_specs=in_specs,
        out_specs=_row_spec(G_PAD, 1),
        out_shape=jax.ShapeDtypeStruct((G_PAD, 1), _f32),
        scratch_shapes=[pltpu.VMEM((_NBG, _BG), _f32)],
    )(x, gid2d, *flat, pred_w, pred_b)


_MESH = plsc.VectorSubcoreMesh(core_axis_name="c", subcore_axis_name="s")
_SC_PARAMS = pltpu.CompilerParams(use_tc_tiling_on_sc=False,
                                  needs_layout_passes=False)
_NC = 2
_NS = 16
_NW = _NC * _NS
_C = 128
_GR = 512


def _gather_rows(table, idx2):
    rows = _GR // _C
    per_w = E_PAD // _NW
    n_chunks = per_w // _GR
    n_pairs = n_chunks // 2

    @functools.partial(
        pl.kernel,
        mesh=_MESH,
        out_type=jax.ShapeDtypeStruct((E_PAD // _C, _C, H), _f32),
        scratch_types=[
            pltpu.VMEM((rows, _C), jnp.int32),
            pltpu.VMEM((rows, _C), jnp.int32),
            pltpu.VMEM((rows, _C, H), _f32),
            pltpu.VMEM((rows, _C, H), _f32),
            pltpu.SemaphoreType.DMA,
            pltpu.SemaphoreType.DMA,
        ],
        compiler_params=_SC_PARAMS,
    )
    def k(table_hbm, idx_hbm, out_hbm, i0, i1, r0, r1, s0, s1):
        wid = lax.axis_index("s") * _NC + lax.axis_index("c")
        ibase = wid * (per_w // _C)
        idx_b = (i0, i1)
        row_b = (r0, r1)
        sem_b = (s0, s1)

        def fire(b, crow):
            pltpu.sync_copy(idx_hbm.at[pl.ds(crow, rows)], idx_b[b])
            for j in range(rows):
                pltpu.async_copy(table_hbm.at[idx_b[b].at[j]],
                                 row_b[b].at[j], sem_b[b])

        def drain(b):
            for j in range(rows):
                pltpu.make_async_copy(table_hbm.at[idx_b[b].at[j]],
                                      row_b[b].at[j], sem_b[b]).wait()

        for b in range(2):
            fire(b, ibase + b * rows)

        def body(i, carry):
            for b in range(2):
                c = 2 * i + b
                crow = ibase + c * rows
                drain(b)
                pltpu.sync_copy(row_b[b], out_hbm.at[pl.ds(crow, rows)])

                @pl.when(c + 2 < n_chunks)
                def _():
                    fire(b, crow + 2 * rows)
            return carry

        lax.fori_loop(0, n_pairs, body, 0)

    return k(table, idx2)


def _seg_softmax_sc(s1_2d, table, idx2, zeros, m_sz, k_sz, rows):
    m_rows = m_sz // _C
    per_sub = m_rows // _NS
    n_chunks = per_sub // rows

    @functools.partial(
        pl.kernel,
        mesh=_MESH,
        out_type=jax.ShapeDtypeStruct((m_rows, _C), _f32),
        scratch_types=[
            pltpu.VMEM_SHARED((k_sz,), _f32),
            pltpu.VMEM((k_sz,), _f32),
            pltpu.VMEM((k_sz,), _f32),
            pltpu.VMEM((rows, _C), jnp.int32),
            pltpu.VMEM((rows, _C), _f32),
            pltpu.VMEM((rows, _C), _f32),
            pltpu.SemaphoreType.DMA,
        ],
        compiler_params=_SC_PARAMS,
    )
    def k(s1_hbm, tab_hbm, idx_hbm, z_hbm, out_hbm,
          spsum, tab_v, sums_v, idx_v, s1_v, w_v, sem):
        c = lax.axis_index("c")
        s = lax.axis_index("s")
        pltpu.sync_copy(tab_hbm, tab_v)

        @pl.when(s == 0)
        def _():
            pltpu.sync_copy(z_hbm, spsum)

        plsc.subcore_barrier()

        def _chunk_w(roff):
            pltpu.sync_copy(idx_hbm.at[pl.ds(roff, rows)], idx_v)
            pltpu.sync_copy(s1_hbm.at[pl.ds(roff, rows)], s1_v)
            for j in range(rows):
                for q in range(_C // 16):
                    sl = pl.ds(q * 16, 16)
                    iv = idx_v[j, sl]
                    tv = plsc.load_gather(tab_v, [iv])
                    sc = s1_v[j, sl] + tv
                    sc = jnp.where(sc >= 0, sc, 0.01 * sc)
                    w_v[j, sl] = jnp.exp(sc)

        def ph1(i, carry):
            roff = s * per_sub + i * rows
            _chunk_w(roff)
            for j in range(rows):
                pltpu.async_copy(w_v.at[j], spsum.at[idx_v.at[j]], sem,
                                 add=True)
            for j in range(rows):
                pltpu.make_async_copy(w_v.at[j], spsum.at[idx_v.at[j]],
                                      sem).wait()
            return carry

        lax.fori_loop(0, n_chunks, ph1, 0)
        plsc.subcore_barrier()
        pltpu.sync_copy(spsum, sums_v)

        def ph2(i, carry):
            roff = s * per_sub + (2 * i + c) * rows
            _chunk_w(roff)
            for j in range(rows):
                for q in range(_C // 16):
                    sl = pl.ds(q * 16, 16)
                    iv = idx_v[j, sl]
                    sg = plsc.load_gather(sums_v, [iv])
                    w_v[j, sl] = w_v[j, sl] / (sg + 1e-12)
            pltpu.sync_copy(w_v, out_hbm.at[pl.ds(roff, rows)])
            return carry

        lax.fori_loop(0, (n_chunks + 1 - c) // 2, ph2, 0)

    return k(s1_2d, table, idx2, zeros)


def _seg_sum32(vals_a, vals_b, idx2, zeros32, m_sz, k_sz, rows):
    chunk = rows * _C
    per_sub = m_sz // _NS
    n_chunks = per_sub // chunk
    rows_out = k_sz // _NS

    @functools.partial(
        pl.kernel,
        mesh=_MESH,
        out_type=[jax.ShapeDtypeStruct((k_sz, H // 2), _f32),
                  jax.ShapeDtypeStruct((k_sz, H // 2), _f32)],
        scratch_types=[
            pltpu.VMEM_SHARED((k_sz, H // 2), _f32),
            pltpu.VMEM((rows, _C), jnp.int32),
            pltpu.VMEM((rows, _C, H // 2), _f32),
            pltpu.SemaphoreType.DMA,
        ],
        compiler_params=_SC_PARAMS,
    )
    def k(va_hbm, vb_hbm, idx_hbm, z_hbm, oa_hbm, ob_hbm,
          spacc, idx_v, vals_v, sem):
        c = lax.axis_index("c")
        s = lax.axis_index("s")

        @pl.when(s == 0)
        def _():
            pltpu.sync_copy(z_hbm, spacc)

        plsc.subcore_barrier()

        def body(i, carry):
            roff = s * (per_sub // _C) + i * rows
            pltpu.sync_copy(idx_hbm.at[pl.ds(roff, rows)], idx_v)

            @pl.when(c == 0)
            def _():
                pltpu.sync_copy(va_hbm.at[pl.ds(roff, rows)], vals_v)

            @pl.when(c == 1)
            def _():
                pltpu.sync_copy(vb_hbm.at[pl.ds(roff, rows)], vals_v)

            for j in range(rows):
                pltpu.async_copy(vals_v.at[j], spacc.at[idx_v.at[j]], sem,
                                 add=True)
            for j in range(rows):
                pltpu.make_async_copy(vals_v.at[j], spacc.at[idx_v.at[j]],
                                      sem).wait()
            return carry

        lax.fori_loop(0, n_chunks, body, 0)
        plsc.subcore_barrier()
        o = s * rows_out

        @pl.when(c == 0)
        def _():
            pltpu.sync_copy(spacc.at[pl.ds(o, rows_out)],
                            oa_hbm.at[pl.ds(o, rows_out)])

        @pl.when(c == 1)
        def _():
            pltpu.sync_copy(spacc.at[pl.ds(o, rows_out)],
                            ob_hbm.at[pl.ds(o, rows_out)])

    return k(vals_a, vals_b, idx2, zeros32)


def kernel(node, edge, edge_index, node_graph_ids, params):
    f32 = _f32
    node_p = jnp.zeros((N_PAD, H), f32).at[:N, :node.shape[1]].set(node)
    edge_p = jnp.zeros((E_PAD, 16), f32).at[:E, :edge.shape[1]].set(edge)
    src = jnp.clip(edge_index[0].astype(jnp.int32), 0, N - 1)
    src2 = jnp.zeros((E_PAD,), jnp.int32).at[:E].set(src).reshape(
        E_PAD // _C, _C)
    dst2 = jnp.full((E_PAD,), N, jnp.int32).at[:E].set(
        edge_index[1].astype(jnp.int32)).reshape(E_PAD // _C, _C)
    gid2d = jnp.full((N_PAD,), G, jnp.int32).at[:N].set(
        node_graph_ids.astype(jnp.int32)).reshape(N_PAD // _BG, _BG)

    zN = jnp.zeros((N_PAD,), f32)
    zN32 = jnp.zeros((N_PAD, H // 2), f32)

    embN_W = jnp.zeros((H, H), f32).at[:, :node.shape[1]].set(params['embN_W'])
    embE_W = jnp.zeros((H, 16), f32).at[:, :edge.shape[1]].set(params['embE_W'])

    embE_b = params['embE_b'].reshape(1, H)

    cA = cB = h = None
    for li, p in enumerate(params['atom']):
        w1 = p['edge_W'][:, :H]
        w2 = p['edge_W'][:, H:]
        a1 = p['align_W'][:, :H]
        a2 = p['align_W'][:, H:]
        ab = p['align_b'].reshape(1, 1)
        nb = p['node_b'].reshape(1, H)
        if li == 0:
            h, hW1, hda = _tc_embnode(node_p, embN_W,
                                      params['embN_b'].reshape(1, H),
                                      p['node_W'], nb, w1, a2)
        else:
            h, hW1, hda = _tc_grunode(cA, cB, h, gprev, p['node_W'], nb,
                                      w1, a2)
        hsrc = _gather_rows(hW1, src2).reshape(E_PAD, H)
        nm, s1 = _tc_edge(hsrc, edge_p, embE_W, embE_b, w2,
                          p['edge_b'].reshape(1, H), a1, ab)
        att = _seg_softmax_sc(s1.reshape(E_PAD // _C, _C),
                              hda.reshape(N_PAD), dst2, zN, E_PAD, N_PAD, 8)
        attcA, attcB = _tc_attc(nm, att.reshape(E_PAD, 1), p['attend_W'],
                                p['attend_b'].reshape(1, H), _BE)
        cA, cB = _seg_sum32(attcA.reshape(E_PAD // _C, _C, H // 2),
                            attcB.reshape(E_PAD // _C, _C, H // 2),
                            dst2, zN32, E_PAD, N_PAD, 4)
        gprev = p['gru']

    x = _tc_gru(cA, cB, h, gprev, _BN)
    out = _tc_molphase(x, gid2d, params['mol'], params['pred_W'],
                       params['pred_b'].reshape(1, 1))
    return out[:G]
